# baseline (speedup 1.0000x reference)
import functools

import jax
import jax.numpy as jnp
from jax import lax
from jax.experimental import pallas as pl
from jax.experimental.pallas import tpu as pltpu
from jax.experimental.pallas import tpu_sc as plsc

N_NODES = 10000
N_EDGES = 320000
D = 128
D_EDGE = 16
N_GRAPHS = 64

NPAD = 10240
N_TILES = 16
N_CORES = 2
ROWS_PER_TILE = NPAD // N_TILES
EDGES_PER_TILE = N_EDGES // (N_CORES * N_TILES)
CHUNK = 40
NCHUNK = EDGES_PER_TILE // CHUNK

BN = 1000
BE = 2000


def _embed_body(at_ref, emb_ref, out_ref):
    at = at_ref[...]
    iota = lax.broadcasted_iota(jnp.int32, (BN, D), 1)
    oh = jnp.equal(at, iota).astype(jnp.float32)
    out_ref[...] = jnp.dot(oh, emb_ref[...], preferred_element_type=jnp.float32)


def _embed(atom2d, emb_pad):
    return pl.pallas_call(
        _embed_body,
        grid=(N_NODES // BN,),
        in_specs=[
            pl.BlockSpec((BN, 1), lambda i: (i, 0)),
            pl.BlockSpec((D, D), lambda i: (0, 0)),
        ],
        out_specs=pl.BlockSpec((BN, D), lambda i: (i, 0)),
        out_shape=jax.ShapeDtypeStruct((N_NODES, D), jnp.float32),
    )(atom2d, emb_pad)


def _edgemlp_body(ea_ref, w1_ref, b1_ref, e1_ref):
    ea = ea_ref[...]
    e = jnp.dot(ea, w1_ref[...], preferred_element_type=jnp.float32) + b1_ref[...]
    e16 = e.astype(jnp.bfloat16).reshape(BE // 2, 2, D)
    lo = lax.bitcast_convert_type(e16[:, 0, :], jnp.uint16).astype(jnp.uint32)
    hi = lax.bitcast_convert_type(e16[:, 1, :], jnp.uint16).astype(jnp.uint32)
    e1_ref[...] = lax.bitcast_convert_type(lo | (hi << 16), jnp.int32)


def _edge_mlp(ea, w1, b1):
    return pl.pallas_call(
        _edgemlp_body,
        grid=(N_EDGES // BE,),
        in_specs=[
            pl.BlockSpec((BE, D_EDGE), lambda i: (i, 0)),
            pl.BlockSpec((D_EDGE, D), lambda i: (0, 0)),
            pl.BlockSpec((1, D), lambda i: (0, 0)),
        ],
        out_specs=pl.BlockSpec((BE // 2, D), lambda i: (i, 0)),
        out_shape=jax.ShapeDtypeStruct((N_EDGES // 2, D), jnp.int32),
    )(ea, w1, b1)


NB = 5


@functools.lru_cache(maxsize=None)
def _make_edge_pass():
    mesh = plsc.VectorSubcoreMesh(core_axis_name="c", subcore_axis_name="s")
    return functools.partial(
        pl.kernel,
        out_type=jax.ShapeDtypeStruct((N_CORES, NPAD, D), jnp.float32),
        mesh=mesh,
        scratch_types=[
            pltpu.VMEM((NB, CHUNK), jnp.int32),
            pltpu.VMEM((NB, CHUNK), jnp.int32),
            pltpu.VMEM((NB * CHUNK // 2 * D,), jnp.int32),
            pltpu.VMEM((NB, CHUNK, D), jnp.float32),
            pltpu.VMEM_SHARED((NPAD, D), jnp.float32),
            pltpu.SemaphoreType.DMA,
            pltpu.SemaphoreType.DMA,
            [pltpu.SemaphoreType.DMA] * NB,
        ],
    )(_edge_pass_body)


def _edge_pass(h, e, src, dst, zrows):
    return _make_edge_pass()(h, e, src, dst, zrows)


def _edge_pass_body(h_hbm, e_hbm, src_hbm, dst_hbm, z_hbm, agg_hbm,
                    si, di, eb, hb, agg_sh, sem_in, sem_g, sem_s):
    c = lax.axis_index("c")
    s = lax.axis_index("s")
    wid = c * N_TILES + s

    slab = pl.ds(s * ROWS_PER_TILE, ROWS_PER_TILE)
    pltpu.sync_copy(z_hbm, agg_sh.at[slab, :])
    plsc.subcore_barrier()

    base = wid * EDGES_PER_TILE

    def start_in(k, b):
        off = base + k * CHUNK
        pltpu.async_copy(src_hbm.at[pl.ds(off, CHUNK)], si.at[b], sem_in)
        pltpu.async_copy(dst_hbm.at[pl.ds(off, CHUNK)], di.at[b], sem_in)
        pltpu.async_copy(e_hbm.at[pl.ds(off * (D // 2), CHUNK * (D // 2))],
                         eb.at[pl.ds(b * CHUNK * (D // 2), CHUNK * (D // 2))],
                         sem_in)

    def wait_in(b):
        pltpu.make_async_copy(src_hbm.at[pl.ds(0, CHUNK)], si.at[b], sem_in).wait()
        pltpu.make_async_copy(dst_hbm.at[pl.ds(0, CHUNK)], di.at[b], sem_in).wait()
        pltpu.make_async_copy(e_hbm.at[pl.ds(0, CHUNK * (D // 2))],
                              eb.at[pl.ds(b * CHUNK * (D // 2), CHUNK * (D // 2))],
                              sem_in).wait()

    def start_gather(b):
        pltpu.async_copy(h_hbm.at[si.at[b]], hb.at[b], sem_g)

    def wait_gather(b):
        pltpu.make_async_copy(h_hbm.at[si.at[b]], hb.at[b], sem_g).wait()

    def start_scatter(b):
        pltpu.async_copy(hb.at[b], agg_sh.at[di.at[b]], sem_s[b], add=True)

    def wait_scatter(b):
        pltpu.make_async_copy(hb.at[b], agg_sh.at[di.at[b]], sem_s[b]).wait()

    def relu_add(b):
        def row_body(rr, carry2):
            r0 = 2 * rr
            for j in range(D // 16):
                sl = pl.ds(j * 16, 16)
                w = eb[pl.ds((b * CHUNK // 2 + rr) * D + j * 16, 16)]
                lo = lax.bitcast_convert_type(w << 16, jnp.float32)
                hi = lax.bitcast_convert_type(w & jnp.int32(-65536), jnp.float32)
                hb[b, r0, sl] = jnp.maximum(hb[b, r0, sl] + lo, 0.0)
                hb[b, r0 + 1, sl] = jnp.maximum(hb[b, r0 + 1, sl] + hi, 0.0)
            return carry2
        lax.fori_loop(0, CHUNK // 2, row_body, 0)

    start_in(0, 0)
    start_in(1, 1)
    wait_in(0)
    start_gather(0)

    def outer(kk, carry):
        for b in range(NB):
            k = kk * NB + b
            s1 = (b + 1) % NB
            s2 = (b + 2) % NB

            @pl.when(k + 1 < NCHUNK)
            def _():
                wait_in(s1)

            @pl.when(k < NCHUNK)
            def _():
                wait_gather(b)

            @pl.when(k + 1 < NCHUNK)
            def _():
                start_gather(s1)

            @pl.when(jnp.logical_and(k >= NB - 2, k - (NB - 2) < NCHUNK))
            def _():
                wait_scatter(s2)

            @pl.when(k + 2 < NCHUNK)
            def _():
                start_in(k + 2, s2)

            @pl.when(k < NCHUNK)
            def _():
                relu_add(b)
                start_scatter(b)
        return carry

    n_outer = (NCHUNK + 2 * NB - 3) // NB
    lax.fori_loop(0, n_outer, outer, 0)

    plsc.subcore_barrier()
    pltpu.sync_copy(agg_sh.at[slab, :], agg_hbm.at[c, slab, :])


def _nodemlp_body(h_ref, a0_ref, a1_ref, w1_ref, b1_ref, w2_ref, b2_ref, out_ref):
    z = h_ref[...] + a0_ref[...].reshape(BN, D) + a1_ref[...].reshape(BN, D)
    z1 = jnp.maximum(jnp.dot(z, w1_ref[...], preferred_element_type=jnp.float32) + b1_ref[...], 0.0)
    z2 = jnp.dot(z1, w2_ref[...], preferred_element_type=jnp.float32) + b2_ref[...]
    out_ref[...] = jnp.tanh(z2)


def _node_mlp_tanh(h, agg, w1, b1, w2, b2):
    return pl.pallas_call(
        _nodemlp_body,
        grid=(N_NODES // BN,),
        in_specs=[
            pl.BlockSpec((BN, D), lambda i: (i, 0)),
            pl.BlockSpec((1, BN, D), lambda i: (0, i, 0)),
            pl.BlockSpec((1, BN, D), lambda i: (1, i, 0)),
            pl.BlockSpec((D, D), lambda i: (0, 0)),
            pl.BlockSpec((1, D), lambda i: (0, 0)),
            pl.BlockSpec((D, D), lambda i: (0, 0)),
            pl.BlockSpec((1, D), lambda i: (0, 0)),
        ],
        out_specs=pl.BlockSpec((BN, D), lambda i: (i, 0)),
        out_shape=jax.ShapeDtypeStruct((N_NODES, D), jnp.float32),
    )(h, agg, agg, w1, b1, w2, b2)


def _pool_body(h_ref, a0_ref, a1_ref, w1_ref, b1_ref, w2_ref, b2_ref, batch_ref,
               out_ref, sums, cnts):
    i = pl.program_id(0)

    @pl.when(i == 0)
    def _():
        sums[...] = jnp.zeros_like(sums)
        cnts[...] = jnp.zeros_like(cnts)

    z = h_ref[...] + a0_ref[...].reshape(BN, D) + a1_ref[...].reshape(BN, D)
    z1 = jnp.maximum(jnp.dot(z, w1_ref[...], preferred_element_type=jnp.float32) + b1_ref[...], 0.0)
    z2 = jnp.dot(z1, w2_ref[...], preferred_element_type=jnp.float32) + b2_ref[...]

    brow = batch_ref[...].reshape(1, BN)
    ohT = jnp.equal(lax.broadcasted_iota(jnp.int32, (N_GRAPHS, BN), 0), brow
                    ).astype(jnp.float32)
    sums[...] += jnp.dot(ohT, z2, preferred_element_type=jnp.float32)
    cnts[...] += jnp.dot(ohT, jnp.ones((BN, D), jnp.float32),
                         preferred_element_type=jnp.float32)

    @pl.when(i == N_NODES // BN - 1)
    def _():
        mean = sums[...] / jnp.maximum(cnts[...], 1.0)
        out_ref[...] = jnp.sum(mean, axis=1, keepdims=True) * (1.0 / D)


def _node_mlp_pool(h, agg, w1, b1, w2, b2, batch3d):
    return pl.pallas_call(
        _pool_body,
        grid=(N_NODES // BN,),
        in_specs=[
            pl.BlockSpec((BN, D), lambda i: (i, 0)),
            pl.BlockSpec((1, BN, D), lambda i: (0, i, 0)),
            pl.BlockSpec((1, BN, D), lambda i: (1, i, 0)),
            pl.BlockSpec((D, D), lambda i: (0, 0)),
            pl.BlockSpec((1, D), lambda i: (0, 0)),
            pl.BlockSpec((D, D), lambda i: (0, 0)),
            pl.BlockSpec((1, D), lambda i: (0, 0)),
            pl.BlockSpec((1, 1, BN), lambda i: (i, 0, 0)),
        ],
        out_specs=pl.BlockSpec((N_GRAPHS, 1), lambda i: (0, 0)),
        out_shape=jax.ShapeDtypeStruct((N_GRAPHS, 1), jnp.float32),
        scratch_shapes=[
            pltpu.VMEM((N_GRAPHS, D), jnp.float32),
            pltpu.VMEM((N_GRAPHS, D), jnp.float32),
        ],
    )(h, agg, agg, w1, b1, w2, b2, batch3d)


def kernel(atom_type, edge_index, edge_attr, batch, atom_emb,
           l1_eW, l1_eb, l1_W1, l1_b1, l1_W2, l1_b2,
           l2_eW, l2_eb, l2_W1, l2_b1, l2_W2, l2_b2):
    src = edge_index[0]
    dst = edge_index[1]
    atom2d = atom_type.reshape(N_NODES, 1)
    batch3d = batch.reshape(N_NODES // BN, 1, BN)
    emb_pad = jnp.pad(atom_emb, ((0, D - atom_emb.shape[0]), (0, 0)))
    zrows = jnp.zeros((ROWS_PER_TILE, D), jnp.float32)

    h0 = _embed(atom2d, emb_pad)
    e1 = _edge_mlp(edge_attr, l1_eW, l1_eb.reshape(1, D))

    agg1 = _edge_pass(h0, e1.reshape(-1), src, dst, zrows)
    e2 = _edge_mlp(edge_attr, l2_eW, l2_eb.reshape(1, D))
    h1 = _node_mlp_tanh(h0, agg1, l1_W1, l1_b1.reshape(1, D),
                        l1_W2, l1_b2.reshape(1, D))

    agg2 = _edge_pass(h1, e2.reshape(-1), src, dst, zrows)
    out = _node_mlp_pool(h1, agg2, l2_W1, l2_b1.reshape(1, D),
                         l2_W2, l2_b2.reshape(1, D), batch3d)
    return out.reshape(N_GRAPHS)


# speedup vs baseline: 1.3674x; 1.3674x over previous
import functools

import jax
import jax.numpy as jnp
from jax import lax
from jax.experimental import pallas as pl
from jax.experimental.pallas import tpu as pltpu
from jax.experimental.pallas import tpu_sc as plsc

N_NODES = 10000
N_EDGES = 320000
D = 128
D_EDGE = 16
N_GRAPHS = 64

NPAD = 10240
N_TILES = 16
N_CORES = 2
ROWS_PER_TILE = NPAD // N_TILES
EDGES_PER_TILE = N_EDGES // (N_CORES * N_TILES)
CHUNK = 40
NCHUNK = (N_EDGES // 2) // (N_CORES * N_TILES) // CHUNK

BN = 1000
BE = 2000


def _embed_body(at_ref, emb_ref, out_ref):
    at = at_ref[...]
    iota = lax.broadcasted_iota(jnp.int32, (BN, D), 1)
    oh = jnp.equal(at, iota).astype(jnp.float32)
    out_ref[...] = jnp.dot(oh, emb_ref[...], preferred_element_type=jnp.float32)


def _embed(atom2d, emb_pad):
    return pl.pallas_call(
        _embed_body,
        grid=(N_NODES // BN,),
        in_specs=[
            pl.BlockSpec((BN, 1), lambda i: (i, 0)),
            pl.BlockSpec((D, D), lambda i: (0, 0)),
        ],
        out_specs=pl.BlockSpec((BN, D), lambda i: (i, 0)),
        out_shape=jax.ShapeDtypeStruct((N_NODES, D), jnp.float32),
    )(atom2d, emb_pad)


BEP = 1000
EHALF = N_EDGES // 2


def _edgemlp_body(ea_lo_ref, ea_hi_ref, w1_ref, b1_ref, e1_ref):
    w1 = w1_ref[...]
    b1 = b1_ref[...]
    e_lo = jnp.dot(ea_lo_ref[...], w1, preferred_element_type=jnp.float32) + b1
    e_hi = jnp.dot(ea_hi_ref[...], w1, preferred_element_type=jnp.float32) + b1
    lo = lax.bitcast_convert_type(e_lo.astype(jnp.bfloat16), jnp.uint16).astype(jnp.uint32)
    hi = lax.bitcast_convert_type(e_hi.astype(jnp.bfloat16), jnp.uint16).astype(jnp.uint32)
    e1_ref[...] = lax.bitcast_convert_type(lo | (hi << 16), jnp.int32)


def _edge_mlp(ea, w1, b1):
    nblk = EHALF // BEP
    return pl.pallas_call(
        _edgemlp_body,
        grid=(nblk,),
        in_specs=[
            pl.BlockSpec((BEP, D_EDGE), lambda i: (i, 0)),
            pl.BlockSpec((BEP, D_EDGE), lambda i, n=nblk: (i + n, 0)),
            pl.BlockSpec((D_EDGE, D), lambda i: (0, 0)),
            pl.BlockSpec((1, D), lambda i: (0, 0)),
        ],
        out_specs=pl.BlockSpec((BEP, D), lambda i: (i, 0)),
        out_shape=jax.ShapeDtypeStruct((EHALF, D), jnp.int32),
    )(ea, ea, w1, b1)


NB = 3


@functools.lru_cache(maxsize=None)
def _make_edge_pass():
    mesh = plsc.VectorSubcoreMesh(core_axis_name="c", subcore_axis_name="s")
    return functools.partial(
        pl.kernel,
        out_type=jax.ShapeDtypeStruct((N_CORES, NPAD, D), jnp.float32),
        mesh=mesh,
        scratch_types=[
            pltpu.VMEM((NB, 2 * CHUNK), jnp.int32),
            pltpu.VMEM((NB, 2 * CHUNK), jnp.int32),
            pltpu.VMEM((NB * CHUNK * D,), jnp.int32),
            pltpu.VMEM((NB, 2 * CHUNK, D), jnp.float32),
            pltpu.VMEM_SHARED((NPAD, D), jnp.float32),
            pltpu.SemaphoreType.DMA,
            pltpu.SemaphoreType.DMA,
            [pltpu.SemaphoreType.DMA] * NB,
        ],
    )(_edge_pass_body)


def _edge_pass(h, e, src, dst, zrows):
    return _make_edge_pass()(h, e, src, dst, zrows)


def _edge_pass_body(h_hbm, e_hbm, src_hbm, dst_hbm, z_hbm, agg_hbm,
                    si, di, eb, hb, agg_sh, sem_in, sem_g, sem_s):
    c = lax.axis_index("c")
    s = lax.axis_index("s")
    wid = c * N_TILES + s

    slab = pl.ds(s * ROWS_PER_TILE, ROWS_PER_TILE)
    pltpu.sync_copy(z_hbm, agg_sh.at[slab, :])
    plsc.subcore_barrier()

    base = wid * (EHALF // (N_CORES * N_TILES))

    def start_in(k, b):
        off = base + k * CHUNK
        pltpu.async_copy(src_hbm.at[pl.ds(off, CHUNK)], si.at[b, pl.ds(0, CHUNK)], sem_in)
        pltpu.async_copy(src_hbm.at[pl.ds(off + EHALF, CHUNK)], si.at[b, pl.ds(CHUNK, CHUNK)], sem_in)
        pltpu.async_copy(dst_hbm.at[pl.ds(off, CHUNK)], di.at[b, pl.ds(0, CHUNK)], sem_in)
        pltpu.async_copy(dst_hbm.at[pl.ds(off + EHALF, CHUNK)], di.at[b, pl.ds(CHUNK, CHUNK)], sem_in)
        pltpu.async_copy(e_hbm.at[pl.ds(off * D, CHUNK * D)],
                         eb.at[pl.ds(b * CHUNK * D, CHUNK * D)], sem_in)

    def wait_in(b):
        for _ in range(4):
            pltpu.make_async_copy(src_hbm.at[pl.ds(0, CHUNK)],
                                  si.at[b, pl.ds(0, CHUNK)], sem_in).wait()
        pltpu.make_async_copy(e_hbm.at[pl.ds(0, CHUNK * D)],
                              eb.at[pl.ds(b * CHUNK * D, CHUNK * D)],
                              sem_in).wait()

    def start_gather(b):
        pltpu.async_copy(h_hbm.at[si.at[b]], hb.at[b], sem_g)

    def wait_gather(b):
        pltpu.make_async_copy(h_hbm.at[si.at[b]], hb.at[b], sem_g).wait()

    def start_scatter(b):
        pltpu.async_copy(hb.at[b], agg_sh.at[di.at[b]], sem_s[b], add=True)

    def wait_scatter(b):
        pltpu.make_async_copy(hb.at[b], agg_sh.at[di.at[b]], sem_s[b]).wait()

    def relu_add(b):
        def row_body(rr, carry2):
            for j in range(D // 16):
                sl = pl.ds(j * 16, 16)
                w = eb[pl.ds((b * CHUNK + rr) * D + j * 16, 16)]
                lo = lax.bitcast_convert_type(w << 16, jnp.float32)
                hi = lax.bitcast_convert_type(w & jnp.int32(-65536), jnp.float32)
                hb[b, rr, sl] = jnp.maximum(hb[b, rr, sl] + lo, 0.0)
                hb[b, rr + CHUNK, sl] = jnp.maximum(hb[b, rr + CHUNK, sl] + hi, 0.0)
            return carry2
        lax.fori_loop(0, CHUNK, row_body, 0)

    start_in(0, 0)
    start_in(1, 1)
    wait_in(0)
    start_gather(0)

    def outer(kk, carry):
        for b in range(NB):
            k = kk * NB + b
            s1 = (b + 1) % NB
            s2 = (b + 2) % NB

            @pl.when(k + 1 < NCHUNK)
            def _():
                wait_in(s1)

            @pl.when(k < NCHUNK)
            def _():
                wait_gather(b)

            @pl.when(k + 1 < NCHUNK)
            def _():
                start_gather(s1)

            @pl.when(jnp.logical_and(k >= NB - 2, k - (NB - 2) < NCHUNK))
            def _():
                wait_scatter(s2)

            @pl.when(k + 2 < NCHUNK)
            def _():
                start_in(k + 2, s2)

            @pl.when(k < NCHUNK)
            def _():
                relu_add(b)
                start_scatter(b)
        return carry

    n_outer = (NCHUNK + 2 * NB - 3) // NB
    lax.fori_loop(0, n_outer, outer, 0)

    plsc.subcore_barrier()
    pltpu.sync_copy(agg_sh.at[slab, :], agg_hbm.at[c, slab, :])


def _nodemlp_body(h_ref, a0_ref, a1_ref, w1_ref, b1_ref, w2_ref, b2_ref, out_ref):
    z = h_ref[...] + a0_ref[...].reshape(BN, D) + a1_ref[...].reshape(BN, D)
    z1 = jnp.maximum(jnp.dot(z, w1_ref[...], preferred_element_type=jnp.float32) + b1_ref[...], 0.0)
    z2 = jnp.dot(z1, w2_ref[...], preferred_element_type=jnp.float32) + b2_ref[...]
    out_ref[...] = jnp.tanh(z2)


def _node_mlp_tanh(h, agg, w1, b1, w2, b2):
    return pl.pallas_call(
        _nodemlp_body,
        grid=(N_NODES // BN,),
        in_specs=[
            pl.BlockSpec((BN, D), lambda i: (i, 0)),
            pl.BlockSpec((1, BN, D), lambda i: (0, i, 0)),
            pl.BlockSpec((1, BN, D), lambda i: (1, i, 0)),
            pl.BlockSpec((D, D), lambda i: (0, 0)),
            pl.BlockSpec((1, D), lambda i: (0, 0)),
            pl.BlockSpec((D, D), lambda i: (0, 0)),
            pl.BlockSpec((1, D), lambda i: (0, 0)),
        ],
        out_specs=pl.BlockSpec((BN, D), lambda i: (i, 0)),
        out_shape=jax.ShapeDtypeStruct((N_NODES, D), jnp.float32),
    )(h, agg, agg, w1, b1, w2, b2)


def _pool_body(h_ref, a0_ref, a1_ref, w1_ref, b1_ref, w2_ref, b2_ref, batch_ref,
               out_ref, sums, cnts):
    i = pl.program_id(0)

    @pl.when(i == 0)
    def _():
        sums[...] = jnp.zeros_like(sums)
        cnts[...] = jnp.zeros_like(cnts)

    z = h_ref[...] + a0_ref[...].reshape(BN, D) + a1_ref[...].reshape(BN, D)
    z1 = jnp.maximum(jnp.dot(z, w1_ref[...], preferred_element_type=jnp.float32) + b1_ref[...], 0.0)
    z2 = jnp.dot(z1, w2_ref[...], preferred_element_type=jnp.float32) + b2_ref[...]

    brow = batch_ref[...].reshape(1, BN)
    ohT = jnp.equal(lax.broadcasted_iota(jnp.int32, (N_GRAPHS, BN), 0), brow
                    ).astype(jnp.float32)
    sums[...] += jnp.dot(ohT, z2, preferred_element_type=jnp.float32)
    cnts[...] += jnp.dot(ohT, jnp.ones((BN, D), jnp.float32),
                         preferred_element_type=jnp.float32)

    @pl.when(i == N_NODES // BN - 1)
    def _():
        mean = sums[...] / jnp.maximum(cnts[...], 1.0)
        out_ref[...] = jnp.sum(mean, axis=1, keepdims=True) * (1.0 / D)


def _node_mlp_pool(h, agg, w1, b1, w2, b2, batch3d):
    return pl.pallas_call(
        _pool_body,
        grid=(N_NODES // BN,),
        in_specs=[
            pl.BlockSpec((BN, D), lambda i: (i, 0)),
            pl.BlockSpec((1, BN, D), lambda i: (0, i, 0)),
            pl.BlockSpec((1, BN, D), lambda i: (1, i, 0)),
            pl.BlockSpec((D, D), lambda i: (0, 0)),
            pl.BlockSpec((1, D), lambda i: (0, 0)),
            pl.BlockSpec((D, D), lambda i: (0, 0)),
            pl.BlockSpec((1, D), lambda i: (0, 0)),
            pl.BlockSpec((1, 1, BN), lambda i: (i, 0, 0)),
        ],
        out_specs=pl.BlockSpec((N_GRAPHS, 1), lambda i: (0, 0)),
        out_shape=jax.ShapeDtypeStruct((N_GRAPHS, 1), jnp.float32),
        scratch_shapes=[
            pltpu.VMEM((N_GRAPHS, D), jnp.float32),
            pltpu.VMEM((N_GRAPHS, D), jnp.float32),
        ],
    )(h, agg, agg, w1, b1, w2, b2, batch3d)


def kernel(atom_type, edge_index, edge_attr, batch, atom_emb,
           l1_eW, l1_eb, l1_W1, l1_b1, l1_W2, l1_b2,
           l2_eW, l2_eb, l2_W1, l2_b1, l2_W2, l2_b2):
    src = edge_index[0]
    dst = edge_index[1]
    atom2d = atom_type.reshape(N_NODES, 1)
    batch3d = batch.reshape(N_NODES // BN, 1, BN)
    emb_pad = jnp.pad(atom_emb, ((0, D - atom_emb.shape[0]), (0, 0)))
    zrows = jnp.zeros((ROWS_PER_TILE, D), jnp.float32)

    h0 = _embed(atom2d, emb_pad)
    e1 = _edge_mlp(edge_attr, l1_eW, l1_eb.reshape(1, D))

    agg1 = _edge_pass(h0, e1.reshape(-1), src, dst, zrows)
    e2 = _edge_mlp(edge_attr, l2_eW, l2_eb.reshape(1, D))
    h1 = _node_mlp_tanh(h0, agg1, l1_W1, l1_b1.reshape(1, D),
                        l1_W2, l1_b2.reshape(1, D))

    agg2 = _edge_pass(h1, e2.reshape(-1), src, dst, zrows)
    out = _node_mlp_pool(h1, agg2, l2_W1, l2_b1.reshape(1, D),
                         l2_W2, l2_b2.reshape(1, D), batch3d)
    return out.reshape(N_GRAPHS)


# speedup vs baseline: 1.7118x; 1.2519x over previous
import functools

import jax
import jax.numpy as jnp
from jax import lax
from jax.experimental import pallas as pl
from jax.experimental.pallas import tpu as pltpu
from jax.experimental.pallas import tpu_sc as plsc

N_NODES = 10000
N_EDGES = 320000
D = 128
D_EDGE = 16
N_GRAPHS = 64

NPAD = 10240
N_TILES = 16
N_CORES = 2
ROWS_PER_TILE = NPAD // N_TILES
EDGES_PER_TILE = N_EDGES // (N_CORES * N_TILES)
CHUNK = 40
NCHUNK = (N_EDGES // 2) // (N_CORES * N_TILES) // CHUNK

BN = 1000
BE = 2000


def _embed_body(at_ref, emb_ref, out_ref):
    at = at_ref[...]
    iota = lax.broadcasted_iota(jnp.int32, (BN, D), 1)
    oh = jnp.equal(at, iota).astype(jnp.float32)
    out_ref[...] = jnp.dot(oh, emb_ref[...], preferred_element_type=jnp.float32)


def _embed(atom2d, emb_pad):
    return pl.pallas_call(
        _embed_body,
        grid=(N_NODES // BN,),
        in_specs=[
            pl.BlockSpec((BN, 1), lambda i: (i, 0)),
            pl.BlockSpec((D, D), lambda i: (0, 0)),
        ],
        out_specs=pl.BlockSpec((BN, D), lambda i: (i, 0)),
        out_shape=jax.ShapeDtypeStruct((N_NODES, D), jnp.float32),
    )(atom2d, emb_pad)


BEP = 1280
EHALF = N_EDGES // 2


_TDN = (((0,), (0,)), ((), ()))


def _edgemlp_body(ea_lo_ref, ea_hi_ref, w1_ref, b1_ref, e1_ref):
    w1 = w1_ref[...]
    b1 = b1_ref[...]
    e_lo = lax.dot_general(ea_lo_ref[...], w1, _TDN,
                           preferred_element_type=jnp.float32) + b1
    e_hi = lax.dot_general(ea_hi_ref[...], w1, _TDN,
                           preferred_element_type=jnp.float32) + b1
    lo = lax.bitcast_convert_type(e_lo.astype(jnp.bfloat16), jnp.uint16).astype(jnp.uint32)
    hi = lax.bitcast_convert_type(e_hi.astype(jnp.bfloat16), jnp.uint16).astype(jnp.uint32)
    e1_ref[...] = lax.bitcast_convert_type(lo | (hi << 16), jnp.int32)


def _edge_mlp(eaT, w1, b1):
    nblk = EHALF // BEP
    return pl.pallas_call(
        _edgemlp_body,
        grid=(nblk,),
        in_specs=[
            pl.BlockSpec((D_EDGE, BEP), lambda i: (0, i)),
            pl.BlockSpec((D_EDGE, BEP), lambda i, n=nblk: (0, i + n)),
            pl.BlockSpec((D_EDGE, D), lambda i: (0, 0)),
            pl.BlockSpec((1, D), lambda i: (0, 0)),
        ],
        out_specs=pl.BlockSpec((BEP, D), lambda i: (i, 0)),
        out_shape=jax.ShapeDtypeStruct((EHALF, D), jnp.int32),
    )(eaT, eaT, w1, b1)


NB = 3


@functools.lru_cache(maxsize=None)
def _make_edge_pass():
    mesh = plsc.VectorSubcoreMesh(core_axis_name="c", subcore_axis_name="s")
    return functools.partial(
        pl.kernel,
        out_type=jax.ShapeDtypeStruct((N_CORES, NPAD, D), jnp.float32),
        mesh=mesh,
        scratch_types=[
            pltpu.VMEM((NB, 2 * CHUNK), jnp.int32),
            pltpu.VMEM((NB, 2 * CHUNK), jnp.int32),
            pltpu.VMEM((NB * CHUNK * D,), jnp.int32),
            pltpu.VMEM((NB, 2 * CHUNK, D), jnp.float32),
            pltpu.VMEM_SHARED((NPAD, D), jnp.float32),
            pltpu.SemaphoreType.DMA,
            pltpu.SemaphoreType.DMA,
            [pltpu.SemaphoreType.DMA] * NB,
        ],
    )(_edge_pass_body)


def _edge_pass(h, e, src, dst, zrows):
    return _make_edge_pass()(h, e, src, dst, zrows)


def _edge_pass_body(h_hbm, e_hbm, src_hbm, dst_hbm, z_hbm, agg_hbm,
                    si, di, eb, hb, agg_sh, sem_in, sem_g, sem_s):
    c = lax.axis_index("c")
    s = lax.axis_index("s")
    wid = c * N_TILES + s

    slab = pl.ds(s * ROWS_PER_TILE, ROWS_PER_TILE)
    pltpu.sync_copy(z_hbm, agg_sh.at[slab, :])
    plsc.subcore_barrier()

    base = wid * (EHALF // (N_CORES * N_TILES))

    def start_in(k, b):
        off = base + k * CHUNK
        pltpu.async_copy(src_hbm.at[pl.ds(off, CHUNK)], si.at[b, pl.ds(0, CHUNK)], sem_in)
        pltpu.async_copy(src_hbm.at[pl.ds(off + EHALF, CHUNK)], si.at[b, pl.ds(CHUNK, CHUNK)], sem_in)
        pltpu.async_copy(dst_hbm.at[pl.ds(off, CHUNK)], di.at[b, pl.ds(0, CHUNK)], sem_in)
        pltpu.async_copy(dst_hbm.at[pl.ds(off + EHALF, CHUNK)], di.at[b, pl.ds(CHUNK, CHUNK)], sem_in)
        pltpu.async_copy(e_hbm.at[pl.ds(off * D, CHUNK * D)],
                         eb.at[pl.ds(b * CHUNK * D, CHUNK * D)], sem_in)

    def wait_in(b):
        for _ in range(4):
            pltpu.make_async_copy(src_hbm.at[pl.ds(0, CHUNK)],
                                  si.at[b, pl.ds(0, CHUNK)], sem_in).wait()
        pltpu.make_async_copy(e_hbm.at[pl.ds(0, CHUNK * D)],
                              eb.at[pl.ds(b * CHUNK * D, CHUNK * D)],
                              sem_in).wait()

    def start_gather(b):
        pltpu.async_copy(h_hbm.at[si.at[b]], hb.at[b], sem_g)

    def wait_gather(b):
        pltpu.make_async_copy(h_hbm.at[si.at[b]], hb.at[b], sem_g).wait()

    def start_scatter(b):
        pltpu.async_copy(hb.at[b], agg_sh.at[di.at[b]], sem_s[b], add=True)

    def wait_scatter(b):
        pltpu.make_async_copy(hb.at[b], agg_sh.at[di.at[b]], sem_s[b]).wait()

    def relu_add(b):
        def row_body(rr, carry2):
            for j in range(D // 16):
                sl = pl.ds(j * 16, 16)
                w = eb[pl.ds((b * CHUNK + rr) * D + j * 16, 16)]
                lo = lax.bitcast_convert_type(w << 16, jnp.float32)
                hi = lax.bitcast_convert_type(w & jnp.int32(-65536), jnp.float32)
                hb[b, rr, sl] = jnp.maximum(hb[b, rr, sl] + lo, 0.0)
                hb[b, rr + CHUNK, sl] = jnp.maximum(hb[b, rr + CHUNK, sl] + hi, 0.0)
            return carry2
        lax.fori_loop(0, CHUNK, row_body, 0)

    start_in(0, 0)
    start_in(1, 1)
    wait_in(0)
    start_gather(0)

    def outer(kk, carry):
        for b in range(NB):
            k = kk * NB + b
            s1 = (b + 1) % NB
            s2 = (b + 2) % NB

            @pl.when(k + 1 < NCHUNK)
            def _():
                wait_in(s1)

            @pl.when(k < NCHUNK)
            def _():
                wait_gather(b)

            @pl.when(k + 1 < NCHUNK)
            def _():
                start_gather(s1)

            @pl.when(jnp.logical_and(k >= NB - 2, k - (NB - 2) < NCHUNK))
            def _():
                wait_scatter(s2)

            @pl.when(k + 2 < NCHUNK)
            def _():
                start_in(k + 2, s2)

            @pl.when(k < NCHUNK)
            def _():
                relu_add(b)
                start_scatter(b)
        return carry

    n_outer = (NCHUNK + 2 * NB - 3) // NB
    lax.fori_loop(0, n_outer, outer, 0)

    plsc.subcore_barrier()
    pltpu.sync_copy(agg_sh.at[slab, :], agg_hbm.at[c, slab, :])


def _nodemlp_body(h_ref, a0_ref, a1_ref, w1_ref, b1_ref, w2_ref, b2_ref, out_ref):
    z = h_ref[...] + a0_ref[...].reshape(BN, D) + a1_ref[...].reshape(BN, D)
    z1 = jnp.maximum(jnp.dot(z, w1_ref[...], preferred_element_type=jnp.float32) + b1_ref[...], 0.0)
    z2 = jnp.dot(z1, w2_ref[...], preferred_element_type=jnp.float32) + b2_ref[...]
    out_ref[...] = jnp.tanh(z2)


def _node_mlp_tanh(h, agg, w1, b1, w2, b2):
    return pl.pallas_call(
        _nodemlp_body,
        grid=(N_NODES // BN,),
        in_specs=[
            pl.BlockSpec((BN, D), lambda i: (i, 0)),
            pl.BlockSpec((1, BN, D), lambda i: (0, i, 0)),
            pl.BlockSpec((1, BN, D), lambda i: (1, i, 0)),
            pl.BlockSpec((D, D), lambda i: (0, 0)),
            pl.BlockSpec((1, D), lambda i: (0, 0)),
            pl.BlockSpec((D, D), lambda i: (0, 0)),
            pl.BlockSpec((1, D), lambda i: (0, 0)),
        ],
        out_specs=pl.BlockSpec((BN, D), lambda i: (i, 0)),
        out_shape=jax.ShapeDtypeStruct((N_NODES, D), jnp.float32),
    )(h, agg, agg, w1, b1, w2, b2)


def _pool_body(h_ref, a0_ref, a1_ref, w1_ref, b1_ref, w2_ref, b2_ref, batch_ref,
               out_ref, sums, cnts):
    i = pl.program_id(0)

    @pl.when(i == 0)
    def _():
        sums[...] = jnp.zeros_like(sums)
        cnts[...] = jnp.zeros_like(cnts)

    z = h_ref[...] + a0_ref[...].reshape(BN, D) + a1_ref[...].reshape(BN, D)
    z1 = jnp.maximum(jnp.dot(z, w1_ref[...], preferred_element_type=jnp.float32) + b1_ref[...], 0.0)
    z2 = jnp.dot(z1, w2_ref[...], preferred_element_type=jnp.float32) + b2_ref[...]

    brow = batch_ref[...].reshape(1, BN)
    ohT = jnp.equal(lax.broadcasted_iota(jnp.int32, (N_GRAPHS, BN), 0), brow
                    ).astype(jnp.float32)
    sums[...] += jnp.dot(ohT, z2, preferred_element_type=jnp.float32)
    cnts[...] += jnp.dot(ohT, jnp.ones((BN, D), jnp.float32),
                         preferred_element_type=jnp.float32)

    @pl.when(i == N_NODES // BN - 1)
    def _():
        mean = sums[...] / jnp.maximum(cnts[...], 1.0)
        out_ref[...] = jnp.sum(mean, axis=1, keepdims=True) * (1.0 / D)


def _node_mlp_pool(h, agg, w1, b1, w2, b2, batch3d):
    return pl.pallas_call(
        _pool_body,
        grid=(N_NODES // BN,),
        in_specs=[
            pl.BlockSpec((BN, D), lambda i: (i, 0)),
            pl.BlockSpec((1, BN, D), lambda i: (0, i, 0)),
            pl.BlockSpec((1, BN, D), lambda i: (1, i, 0)),
            pl.BlockSpec((D, D), lambda i: (0, 0)),
            pl.BlockSpec((1, D), lambda i: (0, 0)),
            pl.BlockSpec((D, D), lambda i: (0, 0)),
            pl.BlockSpec((1, D), lambda i: (0, 0)),
            pl.BlockSpec((1, 1, BN), lambda i: (i, 0, 0)),
        ],
        out_specs=pl.BlockSpec((N_GRAPHS, 1), lambda i: (0, 0)),
        out_shape=jax.ShapeDtypeStruct((N_GRAPHS, 1), jnp.float32),
        scratch_shapes=[
            pltpu.VMEM((N_GRAPHS, D), jnp.float32),
            pltpu.VMEM((N_GRAPHS, D), jnp.float32),
        ],
    )(h, agg, agg, w1, b1, w2, b2, batch3d)


def kernel(atom_type, edge_index, edge_attr, batch, atom_emb,
           l1_eW, l1_eb, l1_W1, l1_b1, l1_W2, l1_b2,
           l2_eW, l2_eb, l2_W1, l2_b1, l2_W2, l2_b2):
    src = edge_index[0]
    dst = edge_index[1]
    atom2d = atom_type.reshape(N_NODES, 1)
    batch3d = batch.reshape(N_NODES // BN, 1, BN)
    emb_pad = jnp.pad(atom_emb, ((0, D - atom_emb.shape[0]), (0, 0)))
    zrows = jnp.zeros((ROWS_PER_TILE, D), jnp.float32)

    eaT = edge_attr.T
    h0 = _embed(atom2d, emb_pad)
    e1 = _edge_mlp(eaT, l1_eW, l1_eb.reshape(1, D))

    agg1 = _edge_pass(h0, e1.reshape(-1), src, dst, zrows)
    e2 = _edge_mlp(eaT, l2_eW, l2_eb.reshape(1, D))
    h1 = _node_mlp_tanh(h0, agg1, l1_W1, l1_b1.reshape(1, D),
                        l1_W2, l1_b2.reshape(1, D))

    agg2 = _edge_pass(h1, e2.reshape(-1), src, dst, zrows)
    out = _node_mlp_pool(h1, agg2, l2_W1, l2_b1.reshape(1, D),
                         l2_W2, l2_b2.reshape(1, D), batch3d)
    return out.reshape(N_GRAPHS)
